# baseline (speedup 1.0000x reference)
import jax
import jax.numpy as jnp
from jax import lax
from jax.experimental import pallas as pl
from jax.experimental.pallas import tpu as pltpu
from jax.experimental.pallas import tpu_sc as plsc

_SHAPE = 100000
_SCALE = 1.0
_DIM = 32
_DX = (_SHAPE - 1) // 2

_B, _S = 4096, 200
_N = _B * _S

_INFO = plsc.get_sparse_core_info()
_NC, _NS, _L = _INFO.num_cores, _INFO.num_subcores, _INFO.num_lanes
_HALF = _SHAPE // _NC
_STAGE = _HALF // _NS
_PER_T = _N // _NS
_SEG = 5120
_NSEG = _PER_T // _SEG
_CH = 1280
_NCH = _SEG // _CH
_UNR = 8
_C = 256
_CAP = _SEG + 2 * _C


def _body(pos_hbm, emb_hbm, out_hbm, shared, pos_v, cidx, opos, rows0, rows1, psem, sem0, sem1, gsem0, gsem1):
    cid = lax.axis_index("c")
    sid = lax.axis_index("s")
    lo = cid * _HALF
    t_base = sid * _PER_T
    lane = lax.iota(jnp.int32, _L)

    pltpu.sync_copy(
        emb_hbm.at[pl.ds(lo + sid * _STAGE, _STAGE)],
        shared.at[pl.ds(sid * _STAGE, _STAGE)],
    )
    plsc.subcore_barrier()

    def pos_load(ch_glob, buf):
        pltpu.async_copy(
            pos_hbm.at[pl.ds(t_base + ch_glob * _CH, _CH)], pos_v.at[buf], psem
        )

    def pos_wait(buf):
        pltpu.make_async_copy(
            pos_hbm.at[pl.ds(t_base, _CH)], pos_v.at[buf], psem
        ).wait()

    pos_load(0, 0)

    for seg in range(_NSEG):
        seg_base = t_base + seg * _SEG

        def chunk(ch, off):
            ch_glob = seg * _NCH + ch
            buf = ch_glob & 1
            pos_wait(buf)

            @pl.when(ch_glob + 1 < _NSEG * _NCH)
            def _():
                pos_load(ch_glob + 1, 1 - buf)

            def grp(gq, off):
                for u in range(_UNR):
                    g = gq * _UNR + u
                    p = pos_v[buf, pl.ds(g * _L, _L)]
                    x = (p * (_DX / _SCALE) + _DX).astype(jnp.int32)
                    x = jnp.clip(x, 0, _SHAPE - 1)
                    loc = x - lo
                    m = (loc >= 0) & (loc < _HALF)
                    gpos = seg_base + ch * _CH + g * _L + lane
                    pre = plsc.cumsum(m.astype(jnp.int32))
                    tgt = jnp.where(m, off + pre - 1, _CAP - _L + lane)
                    plsc.store_scatter(cidx, [tgt], loc)
                    plsc.store_scatter(opos, [tgt], gpos)
                    off = off + plsc.all_reduce_population_count(m)[0]
                return off

            return lax.fori_loop(0, _CH // (_UNR * _L), grp, off)

        off = lax.fori_loop(0, _NCH, chunk, jnp.int32(0))

        @pl.when(off > 0)
        def _():
            last_l = cidx[pl.ds(off - 1, _L)][0]
            last_o = opos[pl.ds(off - 1, _L)][0]
            for q in range(_C // _L):
                cidx[pl.ds(off + q * _L, _L)] = jnp.full((_L,), last_l, jnp.int32)
                opos[pl.ds(off + q * _L, _L)] = jnp.full((_L,), last_o, jnp.int32)

        plsc.subcore_barrier()

        nb = (off + _C - 1) // _C

        def g_start(b, rb, gsem):
            pltpu.async_copy(shared.at[cidx.at[pl.ds(b * _C, _C)]], rb, gsem)

        def g_wait(b, rb, gsem):
            pltpu.make_async_copy(
                shared.at[cidx.at[pl.ds(0, _C)]], rb, gsem
            ).wait()

        def s_start(b, rb, sem):
            pltpu.async_copy(rb, out_hbm.at[opos.at[pl.ds(b * _C, _C)]], sem)

        def s_wait(rb, sem):
            pltpu.make_async_copy(
                rb, out_hbm.at[opos.at[pl.ds(0, _C)]], sem
            ).wait()

        @pl.when(nb > 0)
        def _():
            g_start(0, rows0, gsem0)

        def bat(b, _):
            even = (b & 1) == 0

            @pl.when((b + 1 < nb) & even)
            def _():
                @pl.when(b >= 1)
                def _():
                    s_wait(rows1, sem1)

                g_start(b + 1, rows1, gsem1)

            @pl.when((b + 1 < nb) & jnp.logical_not(even))
            def _():
                s_wait(rows0, sem0)
                g_start(b + 1, rows0, gsem0)

            @pl.when(even)
            def _():
                g_wait(b, rows0, gsem0)
                s_start(b, rows0, sem0)

            @pl.when(jnp.logical_not(even))
            def _():
                g_wait(b, rows1, gsem1)
                s_start(b, rows1, sem1)

            return 0

        lax.fori_loop(0, nb, bat, 0)

        for back in (1, 2):
            @pl.when(nb >= back)
            def _():
                @pl.when(((nb - back) & 1) == 0)
                def _():
                    s_wait(rows0, sem0)

                @pl.when(((nb - back) & 1) == 1)
                def _():
                    s_wait(rows1, sem1)


@jax.jit
def _xembed(pos_flat, embedding):
    mesh = plsc.VectorSubcoreMesh(core_axis_name="c", subcore_axis_name="s")
    return pl.kernel(
        _body,
        mesh=mesh,
        compiler_params=pltpu.CompilerParams(
            use_tc_tiling_on_sc=False, needs_layout_passes=False
        ),
        out_type=jax.ShapeDtypeStruct((_N, _DIM), jnp.float32),
        scratch_types=[
            pltpu.VMEM_SHARED((_HALF, _DIM), jnp.float32),
            pltpu.VMEM((2, _CH), jnp.float32),
            pltpu.VMEM((_CAP,), jnp.int32),
            pltpu.VMEM((_CAP,), jnp.int32),
            pltpu.VMEM((_C, _DIM), jnp.float32),
            pltpu.VMEM((_C, _DIM), jnp.float32),
            pltpu.SemaphoreType.DMA,
            pltpu.SemaphoreType.DMA,
            pltpu.SemaphoreType.DMA,
            pltpu.SemaphoreType.DMA,
            pltpu.SemaphoreType.DMA,
        ],
    )(pos_flat, embedding)


def kernel(pos, embedding):
    out = _xembed(pos.reshape(_N), embedding)
    return out.reshape(_B, _S, _DIM)


# speedup vs baseline: 1.0064x; 1.0064x over previous
import jax
import jax.numpy as jnp
from jax import lax
from jax.experimental import pallas as pl
from jax.experimental.pallas import tpu as pltpu
from jax.experimental.pallas import tpu_sc as plsc

_SHAPE = 100000
_SCALE = 1.0
_DIM = 32
_DX = (_SHAPE - 1) // 2

_B, _S = 4096, 200
_N = _B * _S

_INFO = plsc.get_sparse_core_info()
_NC, _NS, _L = _INFO.num_cores, _INFO.num_subcores, _INFO.num_lanes
_HALF = _SHAPE // _NC
_STAGE = _HALF // _NS
_PER_T = _N // _NS
_SEG = 5120
_NSEG = _PER_T // _SEG
_CH = 640
_NCH = _SEG // _CH
_UNR = 4
_C = 256
_CAP = _SEG + 2 * _C


def _body(pos_hbm, emb_hbm, out_hbm, shared, pos_v, cidx, opos, rows0, rows1, psem, sem0, sem1):
    cid = lax.axis_index("c")
    sid = lax.axis_index("s")
    lo = cid * _HALF
    t_base = sid * _PER_T
    lane = lax.iota(jnp.int32, _L)

    pltpu.sync_copy(
        emb_hbm.at[pl.ds(lo + sid * _STAGE, _STAGE)],
        shared.at[pl.ds(sid * _STAGE, _STAGE)],
    )
    plsc.subcore_barrier()

    def pos_load(ch_glob, buf):
        pltpu.async_copy(
            pos_hbm.at[pl.ds(t_base + ch_glob * _CH, _CH)], pos_v.at[buf], psem
        )

    def pos_wait(buf):
        pltpu.make_async_copy(
            pos_hbm.at[pl.ds(t_base, _CH)], pos_v.at[buf], psem
        ).wait()

    pos_load(0, 0)

    for seg in range(_NSEG):
        seg_base = t_base + seg * _SEG

        def chunk(ch, off):
            ch_glob = seg * _NCH + ch
            buf = ch_glob & 1
            pos_wait(buf)

            @pl.when(ch_glob + 1 < _NSEG * _NCH)
            def _():
                pos_load(ch_glob + 1, 1 - buf)

            def grp(gq, off):
                for u in range(_UNR):
                    g = gq * _UNR + u
                    p = pos_v[buf, pl.ds(g * _L, _L)]
                    x = (p * (_DX / _SCALE) + _DX).astype(jnp.int32)
                    x = jnp.clip(x, 0, _SHAPE - 1)
                    loc = x - lo
                    m = (loc >= 0) & (loc < _HALF)
                    gpos = seg_base + ch * _CH + g * _L + lane
                    pre = plsc.cumsum(m.astype(jnp.int32))
                    tgt = jnp.where(m, off + pre - 1, _CAP - _L + lane)
                    plsc.store_scatter(cidx, [tgt], loc)
                    plsc.store_scatter(opos, [tgt], gpos)
                    off = off + plsc.all_reduce_population_count(m)[0]
                return off

            return lax.fori_loop(0, _CH // (_UNR * _L), grp, off)

        off = lax.fori_loop(0, _NCH, chunk, jnp.int32(0))

        @pl.when(off > 0)
        def _():
            last_l = cidx[pl.ds(off - 1, _L)][0]
            last_o = opos[pl.ds(off - 1, _L)][0]
            for q in range(_C // _L):
                cidx[pl.ds(off + q * _L, _L)] = jnp.full((_L,), last_l, jnp.int32)
                opos[pl.ds(off + q * _L, _L)] = jnp.full((_L,), last_o, jnp.int32)

        plsc.subcore_barrier()

        nb = (off + _C - 1) // _C

        def bat(b, _):
            def fire(rb, sem):
                @pl.when(b >= 2)
                def _():
                    pltpu.make_async_copy(
                        rb, out_hbm.at[opos.at[pl.ds(0, _C)]], sem
                    ).wait()

                pltpu.sync_copy(shared.at[cidx.at[pl.ds(b * _C, _C)]], rb)
                pltpu.async_copy(rb, out_hbm.at[opos.at[pl.ds(b * _C, _C)]], sem)

            even = (b & 1) == 0

            @pl.when(even)
            def _():
                fire(rows0, sem0)

            @pl.when(jnp.logical_not(even))
            def _():
                fire(rows1, sem1)

            return 0

        lax.fori_loop(0, nb, bat, 0)

        for back in (1, 2):
            @pl.when(nb >= back)
            def _():
                @pl.when(((nb - back) & 1) == 0)
                def _():
                    pltpu.make_async_copy(
                        rows0, out_hbm.at[opos.at[pl.ds(0, _C)]], sem0
                    ).wait()

                @pl.when(((nb - back) & 1) == 1)
                def _():
                    pltpu.make_async_copy(
                        rows1, out_hbm.at[opos.at[pl.ds(0, _C)]], sem1
                    ).wait()


@jax.jit
def _xembed(pos_flat, embedding):
    mesh = plsc.VectorSubcoreMesh(core_axis_name="c", subcore_axis_name="s")
    return pl.kernel(
        _body,
        mesh=mesh,
        compiler_params=pltpu.CompilerParams(
            use_tc_tiling_on_sc=False, needs_layout_passes=False
        ),
        out_type=jax.ShapeDtypeStruct((_N, _DIM), jnp.float32),
        scratch_types=[
            pltpu.VMEM_SHARED((_HALF, _DIM), jnp.float32),
            pltpu.VMEM((2, _CH), jnp.float32),
            pltpu.VMEM((_CAP,), jnp.int32),
            pltpu.VMEM((_CAP,), jnp.int32),
            pltpu.VMEM((_C, _DIM), jnp.float32),
            pltpu.VMEM((_C, _DIM), jnp.float32),
            pltpu.SemaphoreType.DMA,
            pltpu.SemaphoreType.DMA,
            pltpu.SemaphoreType.DMA,
        ],
    )(pos_flat, embedding)


def kernel(pos, embedding):
    out = _xembed(pos.reshape(_N), embedding)
    return out.reshape(_B, _S, _DIM)
